# baseline (speedup 1.0000x reference)
import jax
import jax.numpy as jnp
from jax.experimental import pallas as pl
from jax.experimental.pallas import tpu as pltpu


_ROWS = 64
_COLS = 1_000_000
_BLOCK_C = 4096
_LANES = 128
_INT_MAX = 0x7FFFFFFF

_K0 = 0
_K1 = 42
_K2 = _K0 ^ _K1 ^ 0x1BD11BDA

_ROT = ((13, 15, 26, 6), (17, 29, 16, 24))
_ADDS = ((1, 2, 1), (2, 0, 2), (0, 1, 3), (1, 2, 4), (2, 0, 5))


def _threefry_bits(counter):
  ks = (jnp.uint32(_K0), jnp.uint32(_K1), jnp.uint32(_K2))
  x0 = jnp.zeros_like(counter) + ks[0]
  x1 = counter + ks[1]
  for g, (a, b, c) in enumerate(_ADDS):
    for r in _ROT[g % 2]:
      x0 = x0 + x1
      x1 = ((x1 << jnp.uint32(r)) | (x1 >> jnp.uint32(32 - r))) ^ x0
    x0 = x0 + ks[a]
    x1 = x1 + ks[b] + jnp.uint32(c)
  return x0 ^ x1


def _gumbel_from_bits(bits):
  tiny = jnp.float32(jnp.finfo(jnp.float32).tiny)
  float_bits = (bits >> jnp.uint32(9)) | jnp.uint32(0x3F800000)
  floats = jax.lax.bitcast_convert_type(float_bits, jnp.float32) - jnp.float32(1.0)
  u = jnp.maximum(tiny, floats * (jnp.float32(1.0) - tiny) + tiny)
  return -jnp.log(-jnp.log(u))


def _make_body(rows, cols, block_c, grid):
  chunks = block_c // _LANES

  def body(x_ref, out_ref, sval_ref, sidx_ref):
    i = pl.program_id(0)

    @pl.when(i == 0)
    def _init():
      sval_ref[...] = jnp.full((rows, _LANES), -jnp.inf, jnp.float32)
      sidx_ref[...] = jnp.zeros((rows, _LANES), jnp.int32)

    shape = (rows, chunks, _LANES)
    r = jax.lax.broadcasted_iota(jnp.uint32, shape, 0)
    ch = jax.lax.broadcasted_iota(jnp.uint32, shape, 1)
    ln = jax.lax.broadcasted_iota(jnp.uint32, shape, 2)
    col = (jnp.uint32(i) * jnp.uint32(block_c)
           + ch * jnp.uint32(_LANES) + ln)
    counter = r * jnp.uint32(cols) + col

    g = _gumbel_from_bits(_threefry_bits(counter))
    vals = x_ref[...].reshape(shape) + g
    vals = jnp.where(col < jnp.uint32(cols), vals, -jnp.inf)

    m = jnp.max(vals, axis=1)
    coli = col.astype(jnp.int32)
    idx = jnp.min(jnp.where(vals == m[:, None, :], coli, _INT_MAX), axis=1)

    sval = sval_ref[...]
    sidx = sidx_ref[...]
    better = (m > sval) | ((m == sval) & (idx < sidx))
    sval_ref[...] = jnp.where(better, m, sval)
    sidx_ref[...] = jnp.where(better, idx, sidx)

    @pl.when(i == grid - 1)
    def _finish():
      v = sval_ref[...]
      mm = jnp.max(v, axis=1, keepdims=True)
      first = jnp.min(jnp.where(v == mm, sidx_ref[...], _INT_MAX), axis=1)
      out_ref[...] = first.reshape(1, rows)

  return body


def _build(rows, cols, block_c, interpret=False):
  grid = (cols + block_c - 1) // block_c

  def run(x):
    out = pl.pallas_call(
        _make_body(rows, cols, block_c, grid),
        grid=(grid,),
        in_specs=[pl.BlockSpec((rows, block_c), lambda i: (0, i))],
        out_specs=pl.BlockSpec((1, rows), lambda i: (0, 0)),
        out_shape=jax.ShapeDtypeStruct((1, rows), jnp.int32),
        scratch_shapes=[
            pltpu.VMEM((rows, _LANES), jnp.float32),
            pltpu.VMEM((rows, _LANES), jnp.int32),
        ],
        interpret=interpret,
    )(x)
    return out.reshape(rows)

  return run


@jax.jit
def kernel(x):
  return _build(_ROWS, _COLS, _BLOCK_C)(x)


# speedup vs baseline: 1.0109x; 1.0109x over previous
import jax
import jax.numpy as jnp
from jax.experimental import pallas as pl
from jax.experimental.pallas import tpu as pltpu


_ROWS = 64
_COLS = 1_000_000
_BLOCK_C = 4096
_LANES = 128
_INT_MAX = 0x7FFFFFFF

_MAIN_COLS = (_COLS // _BLOCK_C) * _BLOCK_C
_TAIL = _COLS - _MAIN_COLS
_TAIL_CH = (_TAIL + _LANES - 1) // _LANES

_K0 = 0
_K1 = 42
_K2 = _K0 ^ _K1 ^ 0x1BD11BDA

_ROT = ((13, 15, 26, 6), (17, 29, 16, 24))
_ADDS = ((1, 2, 1), (2, 0, 2), (0, 1, 3), (1, 2, 4), (2, 0, 5))


def _threefry_bits(counter):
  ks = (jnp.uint32(_K0), jnp.uint32(_K1), jnp.uint32(_K2))
  x0 = jnp.zeros_like(counter) + ks[0]
  x1 = counter + ks[1]
  for g, (a, b, c) in enumerate(_ADDS):
    for r in _ROT[g % 2]:
      x0 = x0 + x1
      x1 = ((x1 << jnp.uint32(r)) | (x1 >> jnp.uint32(32 - r))) ^ x0
    x0 = x0 + ks[a]
    x1 = x1 + ks[b] + jnp.uint32(c)
  return x0 ^ x1


def _gumbel_from_bits(bits):
  tiny = jnp.float32(jnp.finfo(jnp.float32).tiny)
  float_bits = (bits >> jnp.uint32(9)) | jnp.uint32(0x3F800000)
  floats = jax.lax.bitcast_convert_type(float_bits, jnp.float32) - jnp.float32(1.0)
  u = jnp.maximum(tiny, floats * (jnp.float32(1.0) - tiny) + tiny)
  return -jnp.log(-jnp.log(u))


def _block_vals(rows, cols, shape, col):
  r = jax.lax.broadcasted_iota(jnp.uint32, shape, 0)
  counter = r * jnp.uint32(cols) + col
  return _gumbel_from_bits(_threefry_bits(counter)), col.astype(jnp.int32)


def _chunk_reduce(vals, coli):
  m = jnp.max(vals, axis=1)
  idx = jnp.min(jnp.where(vals == m[:, None, :], coli, _INT_MAX), axis=1)
  return m, idx


def _main_body(rows, cols, block_c, grid):
  chunks = block_c // _LANES

  def body(x_ref, oval_ref, oidx_ref):
    i = pl.program_id(0)

    @pl.when(i == 0)
    def _init():
      oval_ref[...] = jnp.full((rows, _LANES), -jnp.inf, jnp.float32)
      oidx_ref[...] = jnp.zeros((rows, _LANES), jnp.int32)

    shape = (rows, chunks, _LANES)
    ch = jax.lax.broadcasted_iota(jnp.uint32, shape, 1)
    ln = jax.lax.broadcasted_iota(jnp.uint32, shape, 2)
    col = jnp.uint32(i) * jnp.uint32(block_c) + ch * jnp.uint32(_LANES) + ln
    g, coli = _block_vals(rows, cols, shape, col)
    m, idx = _chunk_reduce(x_ref[...].reshape(shape) + g, coli)

    sval = oval_ref[...]
    sidx = oidx_ref[...]
    better = (m > sval) | ((m == sval) & (idx < sidx))
    oval_ref[...] = jnp.where(better, m, sval)
    oidx_ref[...] = jnp.where(better, idx, sidx)

  return body


def _merge_body(rows, cols, tail_start, tail_ch):
  def body(xt_ref, mval_ref, midx_ref, out_ref):
    shape = (rows, tail_ch, _LANES)
    ch = jax.lax.broadcasted_iota(jnp.uint32, shape, 1)
    ln = jax.lax.broadcasted_iota(jnp.uint32, shape, 2)
    col = jnp.uint32(tail_start) + ch * jnp.uint32(_LANES) + ln
    g, coli = _block_vals(rows, cols, shape, col)
    xt = xt_ref[...]
    vals = xt.reshape(shape) + g
    vals = jnp.where(col < jnp.uint32(cols), vals, -jnp.inf)
    tm, tidx = _chunk_reduce(vals, coli)

    mv = mval_ref[...]
    mi = midx_ref[...]
    better = (tm > mv) | ((tm == mv) & (tidx < mi))
    v = jnp.where(better, tm, mv)
    ix = jnp.where(better, tidx, mi)

    mm = jnp.max(v, axis=1, keepdims=True)
    first = jnp.min(jnp.where(v == mm, ix, _INT_MAX), axis=1)
    out_ref[...] = first.reshape(1, rows)

  return body


def _run(x, rows, cols, block_c, interpret=False):
  main_cols = (cols // block_c) * block_c
  grid = main_cols // block_c
  tail = cols - main_cols
  tail_ch = (tail + _LANES - 1) // _LANES

  mval, midx = pl.pallas_call(
      _main_body(rows, cols, block_c, grid),
      grid=(grid,),
      in_specs=[pl.BlockSpec((rows, block_c), lambda i: (0, i))],
      out_specs=[pl.BlockSpec((rows, _LANES), lambda i: (0, 0)),
                 pl.BlockSpec((rows, _LANES), lambda i: (0, 0))],
      out_shape=[jax.ShapeDtypeStruct((rows, _LANES), jnp.float32),
                 jax.ShapeDtypeStruct((rows, _LANES), jnp.int32)],
      interpret=interpret,
  )(x)

  xt = x[:, main_cols:]
  pad = tail_ch * _LANES - tail
  if pad:
    xt = jnp.concatenate(
        [xt, jnp.full((rows, pad), -jnp.inf, jnp.float32)], axis=1)

  out = pl.pallas_call(
      _merge_body(rows, cols, main_cols, tail_ch),
      in_specs=[pl.BlockSpec(xt.shape, lambda: (0, 0)),
                pl.BlockSpec((rows, _LANES), lambda: (0, 0)),
                pl.BlockSpec((rows, _LANES), lambda: (0, 0))],
      out_specs=pl.BlockSpec((1, rows), lambda: (0, 0)),
      out_shape=jax.ShapeDtypeStruct((1, rows), jnp.int32),
      interpret=interpret,
  )(xt, mval, midx)
  return out.reshape(rows)


@jax.jit
def kernel(x):
  return _run(x, _ROWS, _COLS, _BLOCK_C)
